# baseline (speedup 1.0000x reference)
import numpy as np
import jax
import jax.numpy as jnp
from jax import lax
from jax.experimental import pallas as pl
from jax.experimental.pallas import tpu as pltpu
from jax.experimental.pallas import tpu_sc as plsc

VOCAB_N = 1000000
D = 64
BATCH = 4096
SEQ = 200
MAXLEN = 4096

NW = 32
TOTAL = BATCH * SEQ
RPW = TOTAL // NW
C = 128
NCHUNK = RPW // C
NB = 4
NA = 5


def _make_pe2():
    position = np.arange(MAXLEN, dtype=np.float32)[:, None]
    div_term = np.exp(
        np.arange(0, D, 2, dtype=np.float32) * (-np.log(10000.0) / D))
    pe = np.zeros((MAXLEN, D), dtype=np.float32)
    pe[:, 0::2] = np.sin(position * div_term)
    pe[:, 1::2] = np.cos(position * div_term)
    pe = pe[:SEQ]
    return np.concatenate([pe, pe], axis=0)


_PE2 = jnp.asarray(_make_pe2())
_IDENT = jnp.arange(C, dtype=jnp.int32)


def _sc_embed(x3, table, pe2, ident):
    mesh = plsc.VectorSubcoreMesh(core_axis_name="c", subcore_axis_name="s")

    @pl.kernel(
        out_type=jax.ShapeDtypeStruct((TOTAL, D), jnp.float32),
        mesh=mesh,
        compiler_params=pltpu.CompilerParams(use_tc_tiling_on_sc=False),
        scratch_types=[
            pltpu.VMEM((NCHUNK, C), jnp.int32),
            pltpu.VMEM((C, D), jnp.float32),
            pltpu.SemaphoreType.DMA,
        ],
    )
    def k(x_hbm, pe2_hbm, ident_hbm, table_hbm, out_hbm,
          idx_v, rows_v, sem):
        sid = lax.axis_index("s")
        wid = sid * 2 + lax.axis_index("c")
        pltpu.sync_copy(x_hbm.at[wid], idx_v)

        @pl.loop(0, NCHUNK)
        def _(c):
            phase = lax.rem(c * C, SEQ)
            pltpu.sync_copy(pe2_hbm.at[pl.ds(phase, C)], rows_v)
            pltpu.async_copy(table_hbm.at[idx_v.at[c]], rows_v, sem,
                             add=True).wait()
            row0 = wid * RPW + c * C
            pltpu.sync_copy(rows_v, out_hbm.at[pl.ds(row0, C)])

    return k(x3, pe2, ident, table)


def kernel(x, table):
    x3 = x.astype(jnp.int32).reshape(NW, NCHUNK, C)
    out = _sc_embed(x3, table, _PE2, _IDENT)
    return out.reshape(BATCH, SEQ, D)


# speedup vs baseline: 1.0193x; 1.0193x over previous
import numpy as np
import jax
import jax.numpy as jnp
from jax import lax
from jax.experimental import pallas as pl
from jax.experimental.pallas import tpu as pltpu
from jax.experimental.pallas import tpu_sc as plsc

VOCAB_N = 1000000
D = 64
BATCH = 4096
SEQ = 200
MAXLEN = 4096

NW = 32
TOTAL = BATCH * SEQ
RPW = TOTAL // NW
C = 128
NCHUNK = RPW // C
NB = 8
L_GADD = 3
L_STORE = 6


def _make_pe2():
    position = np.arange(MAXLEN, dtype=np.float32)[:, None]
    div_term = np.exp(
        np.arange(0, D, 2, dtype=np.float32) * (-np.log(10000.0) / D))
    pe = np.zeros((MAXLEN, D), dtype=np.float32)
    pe[:, 0::2] = np.sin(position * div_term)
    pe[:, 1::2] = np.cos(position * div_term)
    pe = pe[:SEQ]
    return np.concatenate([pe, pe], axis=0)


_PE2 = jnp.asarray(_make_pe2())


def _sc_embed(x3, table, pe2):
    mesh = plsc.VectorSubcoreMesh(core_axis_name="c", subcore_axis_name="s")

    @pl.kernel(
        out_type=jax.ShapeDtypeStruct((TOTAL, D), jnp.float32),
        mesh=mesh,
        compiler_params=pltpu.CompilerParams(use_tc_tiling_on_sc=False),
        scratch_types=[
            pltpu.VMEM((NCHUNK, C), jnp.int32),
            pltpu.VMEM((NB, C, D), jnp.float32),
            pltpu.SemaphoreType.DMA((NB,)),
            pltpu.SemaphoreType.DMA((NB,)),
            pltpu.SemaphoreType.DMA((NB,)),
        ],
    )
    def k(x_hbm, pe2_hbm, table_hbm, out_hbm,
          idx_v, rows_v, fsem, gsem, ssem):
        sid = lax.axis_index("s")
        wid = sid * 2 + lax.axis_index("c")
        pltpu.sync_copy(x_hbm.at[wid], idx_v)

        def issue_fill(c):
            b = c % NB
            phase = (c * C) % SEQ
            return pltpu.async_copy(pe2_hbm.at[pl.ds(phase, C)],
                                    rows_v.at[b], fsem.at[b])

        def issue_gadd(c):
            b = c % NB
            return pltpu.async_copy(table_hbm.at[idx_v.at[c]],
                                    rows_v.at[b], gsem.at[b], add=True)

        def issue_store(c):
            b = c % NB
            row0 = wid * RPW + c * C
            return pltpu.async_copy(rows_v.at[b],
                                    out_hbm.at[pl.ds(row0, C)], ssem.at[b])

        fills, gadds, stores = {}, {}, {}
        for t in range(NCHUNK + L_STORE):
            c_fill, c_gadd, c_store = t, t - L_GADD, t - L_STORE
            if c_fill < NCHUNK:
                prev = c_fill - NB
                if prev >= 0:
                    stores.pop(prev).wait()
                fills[c_fill] = issue_fill(c_fill)
            if 0 <= c_gadd < NCHUNK:
                fills.pop(c_gadd).wait()
                gadds[c_gadd] = issue_gadd(c_gadd)
            if 0 <= c_store < NCHUNK:
                gadds.pop(c_store).wait()
                stores[c_store] = issue_store(c_store)

        for h in stores.values():
            h.wait()

    return k(x3, pe2, table)


def kernel(x, table):
    x3 = x.astype(jnp.int32).reshape(NW, NCHUNK, C)
    out = _sc_embed(x3, table, _PE2)
    return out.reshape(BATCH, SEQ, D)


# speedup vs baseline: 1.1528x; 1.1310x over previous
import numpy as np
import jax
import jax.numpy as jnp
from jax import lax
from jax.experimental import pallas as pl
from jax.experimental.pallas import tpu as pltpu
from jax.experimental.pallas import tpu_sc as plsc

VOCAB_N = 1000000
D = 64
BATCH = 4096
SEQ = 200
MAXLEN = 4096

NW = 32
TOTAL = BATCH * SEQ
RPW = TOTAL // NW
C = 512
NCHUNK = RPW // C
NB = 3
L_GADD = 1
L_STORE = 2


def _make_pe2():
    position = np.arange(MAXLEN, dtype=np.float32)[:, None]
    div_term = np.exp(
        np.arange(0, D, 2, dtype=np.float32) * (-np.log(10000.0) / D))
    pe = np.zeros((MAXLEN, D), dtype=np.float32)
    pe[:, 0::2] = np.sin(position * div_term)
    pe[:, 1::2] = np.cos(position * div_term)
    pe = pe[:SEQ]
    nrep = (SEQ - 8 + C) // SEQ + 1
    return np.concatenate([pe] * nrep, axis=0)


_PE2 = jnp.asarray(_make_pe2())


def _sc_embed(x3, table, pe2):
    mesh = plsc.VectorSubcoreMesh(core_axis_name="c", subcore_axis_name="s")

    @pl.kernel(
        out_type=jax.ShapeDtypeStruct((TOTAL, D), jnp.float32),
        mesh=mesh,
        compiler_params=pltpu.CompilerParams(use_tc_tiling_on_sc=False),
        scratch_types=[
            pltpu.VMEM((NCHUNK, C), jnp.int32),
            pltpu.VMEM((NB, C, D), jnp.float32),
            pltpu.SemaphoreType.DMA((NB,)),
            pltpu.SemaphoreType.DMA((NB,)),
            pltpu.SemaphoreType.DMA((NB,)),
        ],
    )
    def k(x_hbm, pe2_hbm, table_hbm, out_hbm,
          idx_v, rows_v, fsem, gsem, ssem):
        sid = lax.axis_index("s")
        wid = sid * 2 + lax.axis_index("c")
        pltpu.sync_copy(x_hbm.at[wid], idx_v)

        def issue_fill(c):
            b = c % NB
            phase = (c * C) % SEQ
            return pltpu.async_copy(pe2_hbm.at[pl.ds(phase, C)],
                                    rows_v.at[b], fsem.at[b])

        def issue_gadd(c):
            b = c % NB
            return pltpu.async_copy(table_hbm.at[idx_v.at[c]],
                                    rows_v.at[b], gsem.at[b], add=True)

        def issue_store(c):
            b = c % NB
            row0 = wid * RPW + c * C
            return pltpu.async_copy(rows_v.at[b],
                                    out_hbm.at[pl.ds(row0, C)], ssem.at[b])

        fills, gadds, stores = {}, {}, {}
        for t in range(NCHUNK + L_STORE):
            c_fill, c_gadd, c_store = t, t - L_GADD, t - L_STORE
            if c_fill < NCHUNK:
                prev = c_fill - NB
                if prev >= 0:
                    stores.pop(prev).wait()
                fills[c_fill] = issue_fill(c_fill)
            if 0 <= c_gadd < NCHUNK:
                fills.pop(c_gadd).wait()
                gadds[c_gadd] = issue_gadd(c_gadd)
            if 0 <= c_store < NCHUNK:
                gadds.pop(c_store).wait()
                stores[c_store] = issue_store(c_store)

        for h in stores.values():
            h.wait()

    return k(x3, pe2, table)


def kernel(x, table):
    x3 = x.astype(jnp.int32).reshape(NW, NCHUNK, C)
    out = _sc_embed(x3, table, _PE2)
    return out.reshape(BATCH, SEQ, D)


# speedup vs baseline: 1.3962x; 1.2112x over previous
import numpy as np
import jax
import jax.numpy as jnp
from jax import lax
from jax.experimental import pallas as pl
from jax.experimental.pallas import tpu as pltpu
from jax.experimental.pallas import tpu_sc as plsc

VOCAB_N = 1000000
D = 64
BATCH = 4096
SEQ = 200
MAXLEN = 4096

NW = 32
TOTAL = BATCH * SEQ
RPW = TOTAL // NW
C = 256
NCHUNK = RPW // C
NB = 4
LOOKAHEAD = 2
PE_ROWS = ((SEQ - 8 + C) // SEQ + 1) * SEQ


def _make_pe_rep():
    position = np.arange(MAXLEN, dtype=np.float32)[:, None]
    div_term = np.exp(
        np.arange(0, D, 2, dtype=np.float32) * (-np.log(10000.0) / D))
    pe = np.zeros((MAXLEN, D), dtype=np.float32)
    pe[:, 0::2] = np.sin(position * div_term)
    pe[:, 1::2] = np.cos(position * div_term)
    pe = pe[:SEQ]
    reps = PE_ROWS // SEQ
    return np.concatenate([pe] * reps, axis=0)


_PE_REP = jnp.asarray(_make_pe_rep())


def _sc_embed(x3, table, pe_rep):
    mesh = plsc.VectorSubcoreMesh(core_axis_name="c", subcore_axis_name="s")

    @pl.kernel(
        out_type=jax.ShapeDtypeStruct((TOTAL, D), jnp.float32),
        mesh=mesh,
        compiler_params=pltpu.CompilerParams(use_tc_tiling_on_sc=False),
        scratch_types=[
            pltpu.VMEM((NCHUNK, C), jnp.int32),
            pltpu.VMEM((PE_ROWS, D), jnp.float32),
            pltpu.VMEM((NB, C, D), jnp.float32),
            pltpu.SemaphoreType.DMA((NB,)),
            pltpu.SemaphoreType.DMA((NB,)),
        ],
    )
    def k(x_hbm, pe_hbm, table_hbm, out_hbm,
          idx_v, pe_v, rows_v, gsem, ssem):
        sid = lax.axis_index("s")
        wid = sid * 2 + lax.axis_index("c")
        pltpu.sync_copy(x_hbm.at[wid], idx_v)
        pltpu.sync_copy(pe_hbm, pe_v)

        def issue_gather(c):
            b = c % NB
            return pltpu.async_copy(table_hbm.at[idx_v.at[c]],
                                    rows_v.at[b], gsem.at[b])

        def issue_store(c):
            b = c % NB
            row0 = wid * RPW + c * C
            return pltpu.async_copy(rows_v.at[b],
                                    out_hbm.at[pl.ds(row0, C)], ssem.at[b])

        def add_pe(c):
            b = c % NB
            phase = (c * C) % SEQ
            rv = rows_v.at[b]

            @pl.loop(0, C)
            def _(r):
                for kk in range(D // 16):
                    sl = pl.ds(16 * kk, 16)
                    rv[pl.ds(r, 1), sl] = (rv[pl.ds(r, 1), sl]
                                           + pe_v[pl.ds(phase + r, 1), sl])

        gathers, stores = {}, {}
        for t in range(NCHUNK + LOOKAHEAD):
            if t < NCHUNK:
                prev = t - NB
                if prev >= 0:
                    stores.pop(prev).wait()
                gathers[t] = issue_gather(t)
            c = t - LOOKAHEAD
            if c >= 0:
                gathers.pop(c).wait()
                add_pe(c)
                stores[c] = issue_store(c)

        for h in stores.values():
            h.wait()

    return k(x3, pe_rep, table)


def kernel(x, table):
    x3 = x.astype(jnp.int32).reshape(NW, NCHUNK, C)
    out = _sc_embed(x3, table, _PE_REP)
    return out.reshape(BATCH, SEQ, D)


# speedup vs baseline: 1.4014x; 1.0037x over previous
import numpy as np
import jax
import jax.numpy as jnp
from jax import lax
from jax.experimental import pallas as pl
from jax.experimental.pallas import tpu as pltpu
from jax.experimental.pallas import tpu_sc as plsc

VOCAB_N = 1000000
D = 64
BATCH = 4096
SEQ = 200
MAXLEN = 4096

NW = 32
TOTAL = BATCH * SEQ
RPW = TOTAL // NW
C = SEQ
NCHUNK = RPW // C
NB = 4
LOOKAHEAD = 2
PE_ROWS = SEQ


def _make_pe_rep():
    position = np.arange(MAXLEN, dtype=np.float32)[:, None]
    div_term = np.exp(
        np.arange(0, D, 2, dtype=np.float32) * (-np.log(10000.0) / D))
    pe = np.zeros((MAXLEN, D), dtype=np.float32)
    pe[:, 0::2] = np.sin(position * div_term)
    pe[:, 1::2] = np.cos(position * div_term)
    pe = pe[:SEQ]
    return pe


_PE_REP = jnp.asarray(_make_pe_rep())


def _sc_embed(x3, table, pe_rep):
    mesh = plsc.VectorSubcoreMesh(core_axis_name="c", subcore_axis_name="s")

    @pl.kernel(
        out_type=jax.ShapeDtypeStruct((BATCH, SEQ, D), jnp.float32),
        mesh=mesh,
        compiler_params=pltpu.CompilerParams(use_tc_tiling_on_sc=False),
        scratch_types=[
            pltpu.VMEM((NCHUNK, C), jnp.int32),
            pltpu.VMEM((PE_ROWS, D), jnp.float32),
            pltpu.VMEM((NB, C, D), jnp.float32),
            pltpu.SemaphoreType.DMA((NB,)),
            pltpu.SemaphoreType.DMA((NB,)),
        ],
    )
    def k(x_hbm, pe_hbm, table_hbm, out_hbm,
          idx_v, pe_v, rows_v, gsem, ssem):
        sid = lax.axis_index("s")
        wid = sid * 2 + lax.axis_index("c")
        pltpu.sync_copy(x_hbm.at[wid], idx_v)
        pltpu.sync_copy(pe_hbm, pe_v)

        def issue_gather(c):
            b = c % NB
            return pltpu.async_copy(table_hbm.at[idx_v.at[c]],
                                    rows_v.at[b], gsem.at[b])

        def issue_store(c):
            b = c % NB
            seq_i = wid * NCHUNK + c
            return pltpu.async_copy(rows_v.at[b],
                                    out_hbm.at[seq_i], ssem.at[b])

        def add_pe(c):
            b = c % NB
            rv = rows_v.at[b]

            @pl.loop(0, C)
            def _(r):
                for kk in range(D // 16):
                    sl = pl.ds(16 * kk, 16)
                    rv[pl.ds(r, 1), sl] = (rv[pl.ds(r, 1), sl]
                                           + pe_v[pl.ds(r, 1), sl])

        gathers, stores = {}, {}
        for t in range(NCHUNK + LOOKAHEAD):
            if t < NCHUNK:
                prev = t - NB
                if prev >= 0:
                    stores.pop(prev).wait()
                gathers[t] = issue_gather(t)
            c = t - LOOKAHEAD
            if c >= 0:
                gathers.pop(c).wait()
                add_pe(c)
                stores[c] = issue_store(c)

        for h in stores.values():
            h.wait()

    return k(x3, pe_rep, table)


def kernel(x, table):
    x3 = x.astype(jnp.int32).reshape(NW, NCHUNK, C)
    return _sc_embed(x3, table, _PE_REP)
